# baseline (speedup 1.0000x reference)
import functools

import jax
import jax.numpy as jnp
from jax import lax
from jax.experimental import pallas as pl
from jax.experimental.pallas import tpu as pltpu
from jax.experimental.pallas import tpu_sc as plsc

try:
    _INFO = plsc.get_sparse_core_info()
    _NC, _NS = _INFO.num_cores, _INFO.num_subcores
except Exception:
    _NC, _NS = 2, 16

_CH = 80
_DEGW = 128


def _pad_rows(N):
    return -(-N // (_NS * 8)) * (_NS * 8)


_DEGOUT = 8


def _sc_degree_kernel(N, E):
    NW = _NC * _NS
    epw = E // NW
    n_ch = epw // _CH
    Np = _pad_rows(N)
    rpt = Np // _NS
    mesh = plsc.VectorSubcoreMesh(core_axis_name="c", subcore_axis_name="s")

    @functools.partial(
        pl.kernel,
        out_type=jax.ShapeDtypeStruct((_NC, Np, _DEGW), jnp.float32),
        mesh=mesh,
        scratch_types=[
            pltpu.VMEM((n_ch, _CH), jnp.int32),
            pltpu.VMEM((_CH, _DEGW), jnp.float32),
            pltpu.VMEM((_CH, _DEGW), jnp.float32),
            pltpu.SemaphoreType.DMA,
            pltpu.VMEM_SHARED((Np, _DEGW), jnp.float32),
        ],
    )
    def deg_kernel(dst_hbm, out_hbm, dst_v2, zeros_v, ones_v, dsem, deg_sh):
        cid = lax.axis_index("c")
        sid = lax.axis_index("s")
        wid = sid * _NC + cid

        pltpu.sync_copy(dst_hbm.at[wid], dst_v2)
        zv = jnp.zeros((16,), jnp.float32)
        ov = jnp.ones((16,), jnp.float32)

        def fb(i, _):
            for j in range(_DEGW // 16):
                zeros_v[i, pl.ds(j * 16, 16)] = zv
                ones_v[i, pl.ds(j * 16, 16)] = ov
            return 0

        lax.fori_loop(0, _CH, fb, 0)
        r0 = sid * rpt
        nfull, rem = rpt // _CH, rpt % _CH
        for k in range(nfull):
            pltpu.sync_copy(zeros_v, deg_sh.at[pl.ds(r0 + k * _CH, _CH)])
        if rem:
            pltpu.sync_copy(zeros_v.at[pl.ds(0, rem)],
                            deg_sh.at[pl.ds(r0 + nfull * _CH, rem)])
        plsc.subcore_barrier()

        WIN = 4
        for c in range(WIN):
            pltpu.async_copy(ones_v, deg_sh.at[dst_v2.at[c]], dsem, add=True)

        def body(g, _):
            pltpu.make_async_copy(ones_v, deg_sh.at[dst_v2.at[g - WIN]],
                                  dsem).wait()
            pltpu.async_copy(ones_v, deg_sh.at[dst_v2.at[g]], dsem, add=True)
            return 0

        lax.fori_loop(WIN, n_ch, body, 0)
        for c in range(WIN):
            pltpu.make_async_copy(ones_v,
                                  deg_sh.at[dst_v2.at[n_ch - WIN + c]],
                                  dsem).wait()
        plsc.subcore_barrier()
        pltpu.sync_copy(deg_sh.at[pl.ds(r0, rpt)],
                        out_hbm.at[cid, pl.ds(r0, rpt)])

    return deg_kernel


def _sc_agg_kernel(N, D, E):
    NW = _NC * _NS
    epw = E // NW
    n_ch = epw // _CH
    Np = _pad_rows(N)
    rpt = Np // _NS
    mesh = plsc.VectorSubcoreMesh(core_axis_name="c", subcore_axis_name="s")

    @functools.partial(
        pl.kernel,
        out_type=jax.ShapeDtypeStruct((_NC, Np, D), jnp.float32),
        mesh=mesh,
        scratch_types=[
            pltpu.VMEM((epw,), jnp.int32),
            pltpu.VMEM((n_ch, _CH), jnp.int32),
            pltpu.VMEM((_CH, D), jnp.float32),
            pltpu.VMEM((_CH, D), jnp.float32),
            pltpu.SemaphoreType.DMA,
            pltpu.SemaphoreType.DMA,
            pltpu.SemaphoreType.DMA,
            pltpu.SemaphoreType.DMA,
            pltpu.VMEM_SHARED((Np, D), jnp.float32),
        ],
    )
    def agg_kernel(h_hbm, src_hbm, dst_hbm, out_hbm, src_v1, dst_v2, rows0,
                   rows1, sem0, sem1, ssem0, ssem1, agg_sh):
        cid = lax.axis_index("c")
        sid = lax.axis_index("s")
        wid = sid * _NC + cid

        pltpu.sync_copy(src_hbm.at[pl.ds(wid * epw, epw)], src_v1)
        pltpu.sync_copy(dst_hbm.at[wid], dst_v2)

        def src_at(c):
            return src_v1.at[pl.ds(c * _CH, _CH)]

        zv = jnp.zeros((16,), jnp.float32)

        def zb(i, _):
            for j in range(D // 16):
                rows0[i, pl.ds(j * 16, 16)] = zv
            return 0

        lax.fori_loop(0, _CH, zb, 0)

        r0 = sid * rpt
        nfull, rem = rpt // _CH, rpt % _CH
        for k in range(nfull):
            pltpu.sync_copy(rows0, agg_sh.at[pl.ds(r0 + k * _CH, _CH)])
        if rem:
            pltpu.sync_copy(rows0.at[pl.ds(0, rem)],
                            agg_sh.at[pl.ds(r0 + nfull * _CH, rem)])
        plsc.subcore_barrier()

        half = (n_ch - 1) // 2
        pltpu.async_copy(h_hbm.at[src_at(0)], rows0, sem0)
        pltpu.async_copy(h_hbm.at[src_at(1)], rows1, sem1)

        def body(q, _):
            c0 = 2 * q
            pltpu.make_async_copy(h_hbm.at[src_at(c0)], rows0, sem0).wait()
            pltpu.async_copy(rows0, agg_sh.at[dst_v2.at[c0]], ssem0,
                             add=True)
            pltpu.make_async_copy(h_hbm.at[src_at(c0 + 1)], rows1,
                                  sem1).wait()
            pltpu.make_async_copy(rows0, agg_sh.at[dst_v2.at[c0]],
                                  ssem0).wait()
            pltpu.async_copy(h_hbm.at[src_at(c0 + 2)], rows0, sem0)
            pltpu.async_copy(rows1, agg_sh.at[dst_v2.at[c0 + 1]], ssem1,
                             add=True)
            pltpu.make_async_copy(rows1, agg_sh.at[dst_v2.at[c0 + 1]],
                                  ssem1).wait()

            @pl.when(q < half - 1)
            def _():
                pltpu.async_copy(h_hbm.at[src_at(c0 + 3)], rows1, sem1)

            return 0

        lax.fori_loop(0, half, body, 0)
        pltpu.make_async_copy(h_hbm.at[src_at(n_ch - 1)], rows0,
                              sem0).wait()
        pltpu.sync_copy(rows0, agg_sh.at[dst_v2.at[n_ch - 1]], add=True)
        plsc.subcore_barrier()
        pltpu.sync_copy(agg_sh.at[pl.ds(r0, rpt)],
                        out_hbm.at[cid, pl.ds(r0, rpt)])

    return agg_kernel


def _tc_lin_in(x, W_in, b_in, R=1000):
    N, D = x.shape

    def body(x_ref, w_ref, b_ref, o_ref):
        o_ref[...] = lax.dot_general(
            x_ref[...], w_ref[...], (((1,), (1,)), ((), ())),
            preferred_element_type=jnp.float32) + b_ref[...]

    return pl.pallas_call(
        body,
        grid=(N // R,),
        in_specs=[
            pl.BlockSpec((R, D), lambda i: (i, 0)),
            pl.BlockSpec((D, D), lambda i: (0, 0)),
            pl.BlockSpec((1, D), lambda i: (0, 0)),
        ],
        out_specs=pl.BlockSpec((R, D), lambda i: (i, 0)),
        out_shape=jax.ShapeDtypeStruct((N, D), jnp.float32),
    )(x, W_in, b_in.reshape(1, D))


def _tc_layer(aggp, degp, h, Wl_l, bl_l, Wr_l, gamma, beta, R=1000):
    NC = aggp.shape[0]
    N, D = h.shape
    W = degp.shape[2]

    def body(a_ref, d_ref, h_ref, wl_ref, b_ref, wr_ref, g_ref, be_ref, o_ref):
        a = a_ref[0] + a_ref[1]
        d = d_ref[0, :, 0:1] + d_ref[1, :, 0:1]
        a = a / jnp.maximum(d, 1.0)
        h2 = (lax.dot_general(a, wl_ref[...], (((1,), (1,)), ((), ())),
                              preferred_element_type=jnp.float32)
              + b_ref[...]
              + lax.dot_general(h_ref[...], wr_ref[...], (((1,), (1,)), ((), ())),
                                preferred_element_type=jnp.float32))
        h2 = jnp.maximum(h2, 0.0)
        mu = jnp.mean(h2, axis=1, keepdims=True)
        var = jnp.mean((h2 - mu) ** 2, axis=1, keepdims=True)
        o_ref[...] = (h2 - mu) * lax.rsqrt(var + 1e-5) * g_ref[...] + be_ref[...]

    return pl.pallas_call(
        body,
        grid=(N // R,),
        in_specs=[
            pl.BlockSpec((NC, R, D), lambda i: (0, i, 0)),
            pl.BlockSpec((NC, R, W), lambda i: (0, i, 0)),
            pl.BlockSpec((R, D), lambda i: (i, 0)),
            pl.BlockSpec((D, D), lambda i: (0, 0)),
            pl.BlockSpec((1, D), lambda i: (0, 0)),
            pl.BlockSpec((D, D), lambda i: (0, 0)),
            pl.BlockSpec((1, D), lambda i: (0, 0)),
            pl.BlockSpec((1, D), lambda i: (0, 0)),
        ],
        out_specs=pl.BlockSpec((R, D), lambda i: (i, 0)),
        out_shape=jax.ShapeDtypeStruct((N, D), jnp.float32),
    )(aggp, degp, h, Wl_l, bl_l, Wr_l, gamma, beta)


def kernel(x, edge_index, W_in, b_in, Wl, bl, Wr, gamma, beta):
    N, D = x.shape
    E = edge_index.shape[1]
    L = Wl.shape[0]

    deg_k = _sc_degree_kernel(N, E)
    agg_k = _sc_agg_kernel(N, D, E)

    NW = _NC * _NS
    n_ch = (E // NW) // _CH
    src1 = edge_index[0]
    dst3 = edge_index[1].reshape(NW, n_ch, _CH)
    degp = deg_k(dst3)
    h = _tc_lin_in(x, W_in, b_in)
    g2 = gamma.reshape(1, D)
    b2 = beta.reshape(1, D)
    for l in range(L):
        aggp = agg_k(h, src1, dst3)
        h = _tc_layer(aggp, degp, h, Wl[l], bl[l].reshape(1, D), Wr[l], g2, b2)
    return h


# speedup vs baseline: 1.0122x; 1.0122x over previous
import functools

import jax
import jax.numpy as jnp
from jax import lax
from jax.experimental import pallas as pl
from jax.experimental.pallas import tpu as pltpu
from jax.experimental.pallas import tpu_sc as plsc

try:
    _INFO = plsc.get_sparse_core_info()
    _NC, _NS = _INFO.num_cores, _INFO.num_subcores
except Exception:
    _NC, _NS = 2, 16

_CH = 80
_DEGW = 128


def _pad_rows(N):
    return -(-N // (_NS * 8)) * (_NS * 8)


def _sc_degree_kernel(N, E):
    NW = _NC * _NS
    epw = E // NW
    n_ch = epw // _CH
    Np = _pad_rows(N)
    rpt = Np // _NS
    mesh = plsc.VectorSubcoreMesh(core_axis_name="c", subcore_axis_name="s")

    @functools.partial(
        pl.kernel,
        out_type=jax.ShapeDtypeStruct((_NC, Np, _DEGW), jnp.float32),
        mesh=mesh,
        scratch_types=[
            pltpu.VMEM((n_ch, _CH), jnp.int32),
            pltpu.VMEM((_CH, _DEGW), jnp.float32),
            pltpu.VMEM((_CH, _DEGW), jnp.float32),
            pltpu.VMEM_SHARED((Np, _DEGW), jnp.float32),
        ],
    )
    def deg_kernel(dst_hbm, out_hbm, dst_v2, zeros_v, ones_v, deg_sh):
        cid = lax.axis_index("c")
        sid = lax.axis_index("s")
        wid = sid * _NC + cid

        pltpu.sync_copy(dst_hbm.at[wid], dst_v2)
        zv = jnp.zeros((16,), jnp.float32)
        ov = jnp.ones((16,), jnp.float32)

        def fb(i, _):
            for j in range(_DEGW // 16):
                zeros_v[i, pl.ds(j * 16, 16)] = zv
                ones_v[i, pl.ds(j * 16, 16)] = ov
            return 0

        lax.fori_loop(0, _CH, fb, 0)
        r0 = sid * rpt
        nfull, rem = rpt // _CH, rpt % _CH
        for k in range(nfull):
            pltpu.sync_copy(zeros_v, deg_sh.at[pl.ds(r0 + k * _CH, _CH)])
        if rem:
            pltpu.sync_copy(zeros_v.at[pl.ds(0, rem)],
                            deg_sh.at[pl.ds(r0 + nfull * _CH, rem)])
        plsc.subcore_barrier()

        def body(g, _):
            pltpu.sync_copy(ones_v, deg_sh.at[dst_v2.at[g]], add=True)
            return 0

        lax.fori_loop(0, n_ch, body, 0)
        plsc.subcore_barrier()
        pltpu.sync_copy(deg_sh.at[pl.ds(r0, rpt)],
                        out_hbm.at[cid, pl.ds(r0, rpt)])

    return deg_kernel


def _sc_agg_kernel(N, D, E):
    NW = _NC * _NS
    epw = E // NW
    n_ch = epw // _CH
    Np = _pad_rows(N)
    rpt = Np // _NS
    mesh = plsc.VectorSubcoreMesh(core_axis_name="c", subcore_axis_name="s")

    @functools.partial(
        pl.kernel,
        out_type=jax.ShapeDtypeStruct((_NC, Np, D), jnp.float32),
        mesh=mesh,
        scratch_types=[
            pltpu.VMEM((epw,), jnp.int32),
            pltpu.VMEM((n_ch, _CH), jnp.int32),
            pltpu.VMEM((_CH, D), jnp.float32),
            pltpu.VMEM((_CH, D), jnp.float32),
            pltpu.SemaphoreType.DMA,
            pltpu.SemaphoreType.DMA,
            pltpu.VMEM_SHARED((Np, D), jnp.float32),
        ],
    )
    def agg_kernel(h_hbm, src_hbm, dst_hbm, out_hbm, src_v1, dst_v2, rows0,
                   rows1, sem0, sem1, agg_sh):
        cid = lax.axis_index("c")
        sid = lax.axis_index("s")
        wid = sid * _NC + cid

        pltpu.sync_copy(src_hbm.at[pl.ds(wid * epw, epw)], src_v1)
        pltpu.sync_copy(dst_hbm.at[wid], dst_v2)

        def src_at(c):
            return src_v1.at[pl.ds(c * _CH, _CH)]

        zv = jnp.zeros((16,), jnp.float32)

        def zb(i, _):
            for j in range(D // 16):
                rows0[i, pl.ds(j * 16, 16)] = zv
            return 0

        lax.fori_loop(0, _CH, zb, 0)

        r0 = sid * rpt
        nfull, rem = rpt // _CH, rpt % _CH
        for k in range(nfull):
            pltpu.sync_copy(rows0, agg_sh.at[pl.ds(r0 + k * _CH, _CH)])
        if rem:
            pltpu.sync_copy(rows0.at[pl.ds(0, rem)],
                            agg_sh.at[pl.ds(r0 + nfull * _CH, rem)])
        plsc.subcore_barrier()

        half = (n_ch - 1) // 2
        pltpu.async_copy(h_hbm.at[src_at(0)], rows0, sem0)

        def body(g2, _):
            c0 = 2 * g2
            pltpu.async_copy(h_hbm.at[src_at(c0 + 1)], rows1, sem1)
            pltpu.make_async_copy(h_hbm.at[src_at(c0)], rows0, sem0).wait()
            pltpu.sync_copy(rows0, agg_sh.at[dst_v2.at[c0]], add=True)
            pltpu.async_copy(h_hbm.at[src_at(c0 + 2)], rows0, sem0)
            pltpu.make_async_copy(h_hbm.at[src_at(c0 + 1)], rows1,
                                  sem1).wait()
            pltpu.sync_copy(rows1, agg_sh.at[dst_v2.at[c0 + 1]], add=True)
            return 0

        lax.fori_loop(0, half, body, 0)
        pltpu.make_async_copy(h_hbm.at[src_at(n_ch - 1)], rows0,
                              sem0).wait()
        pltpu.sync_copy(rows0, agg_sh.at[dst_v2.at[n_ch - 1]], add=True)
        plsc.subcore_barrier()
        pltpu.sync_copy(agg_sh.at[pl.ds(r0, rpt)],
                        out_hbm.at[cid, pl.ds(r0, rpt)])

    return agg_kernel


def _tc_lin_in(x, W_in, b_in, R=1000):
    N, D = x.shape

    def body(x_ref, w_ref, b_ref, o_ref):
        o_ref[...] = lax.dot_general(
            x_ref[...], w_ref[...], (((1,), (1,)), ((), ())),
            preferred_element_type=jnp.float32) + b_ref[...]

    return pl.pallas_call(
        body,
        grid=(N // R,),
        in_specs=[
            pl.BlockSpec((R, D), lambda i: (i, 0)),
            pl.BlockSpec((D, D), lambda i: (0, 0)),
            pl.BlockSpec((1, D), lambda i: (0, 0)),
        ],
        out_specs=pl.BlockSpec((R, D), lambda i: (i, 0)),
        out_shape=jax.ShapeDtypeStruct((N, D), jnp.float32),
    )(x, W_in, b_in.reshape(1, D))


def _tc_layer(aggp, degp, h, Wl_l, bl_l, Wr_l, gamma, beta, R=1000):
    NC = aggp.shape[0]
    N, D = h.shape
    W = degp.shape[2]

    def body(a_ref, d_ref, h_ref, wl_ref, b_ref, wr_ref, g_ref, be_ref, o_ref):
        a = a_ref[0] + a_ref[1]
        d = d_ref[0, :, 0:1] + d_ref[1, :, 0:1]
        a = a / jnp.maximum(d, 1.0)
        h2 = (lax.dot_general(a, wl_ref[...], (((1,), (1,)), ((), ())),
                              preferred_element_type=jnp.float32)
              + b_ref[...]
              + lax.dot_general(h_ref[...], wr_ref[...], (((1,), (1,)), ((), ())),
                                preferred_element_type=jnp.float32))
        h2 = jnp.maximum(h2, 0.0)
        mu = jnp.mean(h2, axis=1, keepdims=True)
        var = jnp.mean((h2 - mu) ** 2, axis=1, keepdims=True)
        o_ref[...] = (h2 - mu) * lax.rsqrt(var + 1e-5) * g_ref[...] + be_ref[...]

    return pl.pallas_call(
        body,
        grid=(N // R,),
        in_specs=[
            pl.BlockSpec((NC, R, D), lambda i: (0, i, 0)),
            pl.BlockSpec((NC, R, W), lambda i: (0, i, 0)),
            pl.BlockSpec((R, D), lambda i: (i, 0)),
            pl.BlockSpec((D, D), lambda i: (0, 0)),
            pl.BlockSpec((1, D), lambda i: (0, 0)),
            pl.BlockSpec((D, D), lambda i: (0, 0)),
            pl.BlockSpec((1, D), lambda i: (0, 0)),
            pl.BlockSpec((1, D), lambda i: (0, 0)),
        ],
        out_specs=pl.BlockSpec((R, D), lambda i: (i, 0)),
        out_shape=jax.ShapeDtypeStruct((N, D), jnp.float32),
    )(aggp, degp, h, Wl_l, bl_l, Wr_l, gamma, beta)


def kernel(x, edge_index, W_in, b_in, Wl, bl, Wr, gamma, beta):
    N, D = x.shape
    E = edge_index.shape[1]
    L = Wl.shape[0]

    deg_k = _sc_degree_kernel(N, E)
    agg_k = _sc_agg_kernel(N, D, E)

    NW = _NC * _NS
    n_ch = (E // NW) // _CH
    src1 = edge_index[0]
    dst3 = edge_index[1].reshape(NW, n_ch, _CH)
    degp = deg_k(dst3)
    h = _tc_lin_in(x, W_in, b_in)
    g2 = gamma.reshape(1, D)
    b2 = beta.reshape(1, D)
    for l in range(L):
        aggp = agg_k(h, src1, dst3)
        h = _tc_layer(aggp, degp, h, Wl[l], bl[l].reshape(1, D), Wr[l], g2, b2)
    return h


# speedup vs baseline: 1.0164x; 1.0041x over previous
import functools

import jax
import jax.numpy as jnp
from jax import lax
from jax.experimental import pallas as pl
from jax.experimental.pallas import tpu as pltpu
from jax.experimental.pallas import tpu_sc as plsc

try:
    _INFO = plsc.get_sparse_core_info()
    _NC, _NS = _INFO.num_cores, _INFO.num_subcores
except Exception:
    _NC, _NS = 2, 16

_CH = 80
_DEGW = 128


def _pad_rows(N):
    return -(-N // (_NS * 8)) * (_NS * 8)


def _sc_degree_kernel(N, E):
    NW = _NC * _NS
    epw = E // NW
    n_ch = epw // _CH
    Np = _pad_rows(N)
    rpt = Np // _NS
    mesh = plsc.VectorSubcoreMesh(core_axis_name="c", subcore_axis_name="s")

    @functools.partial(
        pl.kernel,
        out_type=jax.ShapeDtypeStruct((_NC, Np, _DEGW), jnp.float32),
        mesh=mesh,
        scratch_types=[
            pltpu.VMEM((n_ch, _CH), jnp.int32),
            pltpu.VMEM((_CH, _DEGW), jnp.float32),
            pltpu.VMEM((_CH, _DEGW), jnp.float32),
            pltpu.SemaphoreType.DMA,
            pltpu.VMEM_SHARED((Np, _DEGW), jnp.float32),
        ],
    )
    def deg_kernel(dst_hbm, out_hbm, dst_v2, zeros_v, ones_v, dsem, deg_sh):
        cid = lax.axis_index("c")
        sid = lax.axis_index("s")
        wid = sid * _NC + cid

        pltpu.sync_copy(dst_hbm.at[wid], dst_v2)
        zv = jnp.zeros((16,), jnp.float32)
        ov = jnp.ones((16,), jnp.float32)

        def fb(i, _):
            for j in range(_DEGW // 16):
                zeros_v[i, pl.ds(j * 16, 16)] = zv
                ones_v[i, pl.ds(j * 16, 16)] = ov
            return 0

        lax.fori_loop(0, _CH, fb, 0)
        r0 = sid * rpt
        nfull, rem = rpt // _CH, rpt % _CH
        for k in range(nfull):
            pltpu.sync_copy(zeros_v, deg_sh.at[pl.ds(r0 + k * _CH, _CH)])
        if rem:
            pltpu.sync_copy(zeros_v.at[pl.ds(0, rem)],
                            deg_sh.at[pl.ds(r0 + nfull * _CH, rem)])
        plsc.subcore_barrier()

        WIN = 4
        for c in range(WIN):
            pltpu.async_copy(ones_v, deg_sh.at[dst_v2.at[c]], dsem, add=True)

        def body(g, _):
            pltpu.make_async_copy(ones_v, deg_sh.at[dst_v2.at[g - WIN]],
                                  dsem).wait()
            pltpu.async_copy(ones_v, deg_sh.at[dst_v2.at[g]], dsem, add=True)
            return 0

        lax.fori_loop(WIN, n_ch, body, 0)
        for c in range(WIN):
            pltpu.make_async_copy(ones_v,
                                  deg_sh.at[dst_v2.at[n_ch - WIN + c]],
                                  dsem).wait()
        plsc.subcore_barrier()
        pltpu.sync_copy(deg_sh.at[pl.ds(r0, rpt)],
                        out_hbm.at[cid, pl.ds(r0, rpt)])

    return deg_kernel


def _sc_agg_kernel(N, D, E):
    NW = _NC * _NS
    epw = E // NW
    n_ch = epw // _CH
    Np = _pad_rows(N)
    rpt = Np // _NS
    mesh = plsc.VectorSubcoreMesh(core_axis_name="c", subcore_axis_name="s")

    @functools.partial(
        pl.kernel,
        out_type=jax.ShapeDtypeStruct((_NC, Np, D), jnp.float32),
        mesh=mesh,
        scratch_types=[
            pltpu.VMEM((epw,), jnp.int32),
            pltpu.VMEM((n_ch, _CH), jnp.int32),
            pltpu.VMEM((_CH, D), jnp.float32),
            pltpu.VMEM((_CH, D), jnp.float32),
            pltpu.SemaphoreType.DMA,
            pltpu.SemaphoreType.DMA,
            pltpu.VMEM_SHARED((Np, D), jnp.float32),
        ],
    )
    def agg_kernel(h_hbm, src_hbm, dst_hbm, out_hbm, src_v1, dst_v2, rows0,
                   rows1, sem0, sem1, agg_sh):
        cid = lax.axis_index("c")
        sid = lax.axis_index("s")
        wid = sid * _NC + cid

        pltpu.sync_copy(src_hbm.at[pl.ds(wid * epw, epw)], src_v1)
        pltpu.sync_copy(dst_hbm.at[wid], dst_v2)

        def src_at(c):
            return src_v1.at[pl.ds(c * _CH, _CH)]

        zv = jnp.zeros((16,), jnp.float32)

        def zb(i, _):
            for j in range(D // 16):
                rows0[i, pl.ds(j * 16, 16)] = zv
            return 0

        lax.fori_loop(0, _CH, zb, 0)

        r0 = sid * rpt
        nfull, rem = rpt // _CH, rpt % _CH
        for k in range(nfull):
            pltpu.sync_copy(rows0, agg_sh.at[pl.ds(r0 + k * _CH, _CH)])
        if rem:
            pltpu.sync_copy(rows0.at[pl.ds(0, rem)],
                            agg_sh.at[pl.ds(r0 + nfull * _CH, rem)])
        plsc.subcore_barrier()

        half = (n_ch - 1) // 2
        pltpu.async_copy(h_hbm.at[src_at(0)], rows0, sem0)

        def body(g2, _):
            c0 = 2 * g2
            pltpu.async_copy(h_hbm.at[src_at(c0 + 1)], rows1, sem1)
            pltpu.make_async_copy(h_hbm.at[src_at(c0)], rows0, sem0).wait()
            pltpu.sync_copy(rows0, agg_sh.at[dst_v2.at[c0]], add=True)
            pltpu.async_copy(h_hbm.at[src_at(c0 + 2)], rows0, sem0)
            pltpu.make_async_copy(h_hbm.at[src_at(c0 + 1)], rows1,
                                  sem1).wait()
            pltpu.sync_copy(rows1, agg_sh.at[dst_v2.at[c0 + 1]], add=True)
            return 0

        lax.fori_loop(0, half, body, 0)
        pltpu.make_async_copy(h_hbm.at[src_at(n_ch - 1)], rows0,
                              sem0).wait()
        pltpu.sync_copy(rows0, agg_sh.at[dst_v2.at[n_ch - 1]], add=True)
        plsc.subcore_barrier()
        pltpu.sync_copy(agg_sh.at[pl.ds(r0, rpt)],
                        out_hbm.at[cid, pl.ds(r0, rpt)])

    return agg_kernel


def _tc_lin_in(x, W_in, b_in, R=1000):
    N, D = x.shape

    def body(x_ref, w_ref, b_ref, o_ref):
        o_ref[...] = lax.dot_general(
            x_ref[...], w_ref[...], (((1,), (1,)), ((), ())),
            preferred_element_type=jnp.float32) + b_ref[...]

    return pl.pallas_call(
        body,
        grid=(N // R,),
        in_specs=[
            pl.BlockSpec((R, D), lambda i: (i, 0)),
            pl.BlockSpec((D, D), lambda i: (0, 0)),
            pl.BlockSpec((1, D), lambda i: (0, 0)),
        ],
        out_specs=pl.BlockSpec((R, D), lambda i: (i, 0)),
        out_shape=jax.ShapeDtypeStruct((N, D), jnp.float32),
    )(x, W_in, b_in.reshape(1, D))


def _tc_layer(aggp, degp, h, Wl_l, bl_l, Wr_l, gamma, beta, R=1000):
    NC = aggp.shape[0]
    N, D = h.shape
    W = degp.shape[2]

    def body(a_ref, d_ref, h_ref, wl_ref, b_ref, wr_ref, g_ref, be_ref, o_ref):
        a = a_ref[0] + a_ref[1]
        d = d_ref[0, :, 0:1] + d_ref[1, :, 0:1]
        a = a / jnp.maximum(d, 1.0)
        h2 = (lax.dot_general(a, wl_ref[...], (((1,), (1,)), ((), ())),
                              preferred_element_type=jnp.float32)
              + b_ref[...]
              + lax.dot_general(h_ref[...], wr_ref[...], (((1,), (1,)), ((), ())),
                                preferred_element_type=jnp.float32))
        h2 = jnp.maximum(h2, 0.0)
        mu = jnp.mean(h2, axis=1, keepdims=True)
        var = jnp.mean((h2 - mu) ** 2, axis=1, keepdims=True)
        o_ref[...] = (h2 - mu) * lax.rsqrt(var + 1e-5) * g_ref[...] + be_ref[...]

    return pl.pallas_call(
        body,
        grid=(N // R,),
        in_specs=[
            pl.BlockSpec((NC, R, D), lambda i: (0, i, 0)),
            pl.BlockSpec((NC, R, W), lambda i: (0, i, 0)),
            pl.BlockSpec((R, D), lambda i: (i, 0)),
            pl.BlockSpec((D, D), lambda i: (0, 0)),
            pl.BlockSpec((1, D), lambda i: (0, 0)),
            pl.BlockSpec((D, D), lambda i: (0, 0)),
            pl.BlockSpec((1, D), lambda i: (0, 0)),
            pl.BlockSpec((1, D), lambda i: (0, 0)),
        ],
        out_specs=pl.BlockSpec((R, D), lambda i: (i, 0)),
        out_shape=jax.ShapeDtypeStruct((N, D), jnp.float32),
    )(aggp, degp, h, Wl_l, bl_l, Wr_l, gamma, beta)


def kernel(x, edge_index, W_in, b_in, Wl, bl, Wr, gamma, beta):
    N, D = x.shape
    E = edge_index.shape[1]
    L = Wl.shape[0]

    deg_k = _sc_degree_kernel(N, E)
    agg_k = _sc_agg_kernel(N, D, E)

    NW = _NC * _NS
    n_ch = (E // NW) // _CH
    src1 = edge_index[0]
    dst3 = edge_index[1].reshape(NW, n_ch, _CH)
    degp = deg_k(dst3)
    h = _tc_lin_in(x, W_in, b_in)
    g2 = gamma.reshape(1, D)
    b2 = beta.reshape(1, D)
    for l in range(L):
        aggp = agg_k(h, src1, dst3)
        h = _tc_layer(aggp, degp, h, Wl[l], bl[l].reshape(1, D), Wr[l], g2, b2)
    return h
